# Initial kernel scaffold; baseline (speedup 1.0000x reference)
import functools

import jax
import jax.numpy as jnp
from jax import lax
from jax.experimental import pallas as pl
from jax.experimental.pallas import tpu as pltpu
from jax.experimental.pallas import tpu_sc as plsc

N = 10000
E = 320000
D = 128
EPS = 1e-5

NC = 2
NS = 16
NW = NC * NS

CHUNK = 80
EDGES_PER_W = E // NW
CHUNKS_PER_W = EDGES_PER_W // CHUNK
ROWS_PER_TILE = N // NS

_MESH = plsc.VectorSubcoreMesh(
    core_axis_name="c", subcore_axis_name="s", num_cores=NC, num_subcores=NS
)


@functools.partial(
    pl.kernel,
    out_type=jax.ShapeDtypeStruct((NC, N, 16), jnp.float32),
    mesh=_MESH,
    scratch_types=[
        pltpu.VMEM((CHUNK,), jnp.int32),
        pltpu.VMEM((CHUNK, 16), jnp.float32),
        pltpu.VMEM_SHARED((N, 16), jnp.float32),
    ],
)
def _sc_count(edge_hbm, zeros_hbm, ones_hbm, out_hbm, idx_v, ones_v, cnt_sh):
    c = lax.axis_index("c")
    s = lax.axis_index("s")
    row0 = pl.multiple_of(s * ROWS_PER_TILE, 8)
    pltpu.sync_copy(zeros_hbm.at[pl.ds(row0, ROWS_PER_TILE)],
                    cnt_sh.at[pl.ds(row0, ROWS_PER_TILE)])
    pltpu.sync_copy(ones_hbm, ones_v)
    plsc.subcore_barrier()

    per_tile = E // NS

    def chunk(j, _):
        base = pl.multiple_of(s * per_tile + j * CHUNK, 8)
        pltpu.sync_copy(edge_hbm.at[c, pl.ds(base, CHUNK)], idx_v)
        pltpu.sync_copy(ones_v, cnt_sh.at[idx_v], add=True)
        return ()

    lax.fori_loop(0, per_tile // CHUNK, chunk, ())
    plsc.subcore_barrier()
    pltpu.sync_copy(cnt_sh.at[pl.ds(row0, ROWS_PER_TILE)],
                    out_hbm.at[c, pl.ds(row0, ROWS_PER_TILE)])


@functools.partial(
    pl.kernel,
    out_type=jax.ShapeDtypeStruct((NC, N, D), jnp.float32),
    mesh=_MESH,
    scratch_types=[
        pltpu.VMEM((CHUNK,), jnp.int32),
        pltpu.VMEM((CHUNK,), jnp.int32),
        pltpu.VMEM((CHUNK, D), jnp.float32),
        pltpu.VMEM_SHARED((N, D), jnp.float32),
        pltpu.SemaphoreType.DMA,
    ],
)
def _sc_agg(q_hbm, edge_hbm, zeros_hbm, out_hbm, sidx, didx, rows, acc_sh, sem):
    c = lax.axis_index("c")
    s = lax.axis_index("s")
    row0 = pl.multiple_of(s * ROWS_PER_TILE, 8)
    pltpu.sync_copy(zeros_hbm.at[pl.ds(row0, ROWS_PER_TILE)],
                    acc_sh.at[pl.ds(row0, ROWS_PER_TILE)])
    plsc.subcore_barrier()

    ebase = (c * NS + s) * EDGES_PER_W

    def chunk(j, _):
        base = pl.multiple_of(ebase + j * CHUNK, 8)
        pltpu.sync_copy(edge_hbm.at[0, pl.ds(base, CHUNK)], sidx)
        pltpu.sync_copy(edge_hbm.at[1, pl.ds(base, CHUNK)], didx)
        pltpu.async_copy(q_hbm.at[sidx], rows, sem).wait()
        pltpu.sync_copy(rows, acc_sh.at[didx], add=True)
        return ()

    lax.fori_loop(0, CHUNKS_PER_W, chunk, ())
    plsc.subcore_barrier()
    pltpu.sync_copy(acc_sh.at[pl.ds(row0, ROWS_PER_TILE)],
                    out_hbm.at[c, pl.ds(row0, ROWS_PER_TILE)])


def _inv_sqrt_deg(cnt_ref, which):
    c = cnt_ref[which, :, 0:1]
    return lax.rsqrt(jnp.maximum(c, 1.0))


def _tc_first_body(feat_ref, w_ref, cnt_ref, q_ref):
    inv_out = _inv_sqrt_deg(cnt_ref, 0)
    q_ref[...] = jnp.dot(feat_ref[...], w_ref[...],
                         preferred_element_type=jnp.float32) * inv_out


def _tc_mid_body(gp_ref, cnt_ref, gamma_ref, beta_ref, w_ref, q_ref, *,
                 matmul):
    inv_in = _inv_sqrt_deg(cnt_ref, 1)
    inv_out = _inv_sqrt_deg(cnt_ref, 0)
    z = (gp_ref[0] + gp_ref[1]) * inv_in
    m = jnp.mean(z, axis=0, keepdims=True)
    d = z - m
    v = jnp.mean(d * d, axis=0, keepdims=True)
    h = d * lax.rsqrt(v + EPS) * gamma_ref[...] + beta_ref[...]
    h = jnp.maximum(h, 0.0)
    if matmul:
        h = jnp.dot(h, w_ref[...], preferred_element_type=jnp.float32)
    q_ref[...] = h * inv_out


def _tc_last_body(gp_ref, cnt_ref, w_ref, b_ref, out_ref):
    inv_in = _inv_sqrt_deg(cnt_ref, 1)
    z = (gp_ref[0] + gp_ref[1]) * inv_in
    out_ref[...] = jnp.dot(z, w_ref[...],
                           preferred_element_type=jnp.float32) + b_ref[...]


_tc_first = pl.pallas_call(
    _tc_first_body, out_shape=jax.ShapeDtypeStruct((N, D), jnp.float32))

_tc_mid = pl.pallas_call(
    functools.partial(_tc_mid_body, matmul=True),
    out_shape=jax.ShapeDtypeStruct((N, D), jnp.float32))

_tc_premul = pl.pallas_call(
    functools.partial(_tc_mid_body, matmul=False),
    out_shape=jax.ShapeDtypeStruct((N, D), jnp.float32))

_tc_last = pl.pallas_call(
    _tc_last_body, out_shape=jax.ShapeDtypeStruct((N, 40), jnp.float32))


def kernel(feat, edge_index, W0, W1, W2, b2, gamma0, beta0, gamma1, beta1):
    zeros16 = jnp.zeros((N, 16), jnp.float32)
    ones16 = jnp.ones((CHUNK, 16), jnp.float32)
    zerosD = jnp.zeros((N, D), jnp.float32)

    cnt = _sc_count(edge_index, zeros16, ones16)
    q0 = _tc_first(feat, W0, cnt)
    g0 = _sc_agg(q0, edge_index, zerosD)
    q1 = _tc_mid(g0, cnt, gamma0, beta0, W1)
    g1 = _sc_agg(q1, edge_index, zerosD)
    y2 = _tc_premul(g1, cnt, gamma1, beta1, W1)
    g2 = _sc_agg(y2, edge_index, zerosD)
    out = _tc_last(g2, cnt, W2, b2)
    return out


# speedup vs baseline: 2.8881x; 2.8881x over previous
import functools

import jax
import jax.numpy as jnp
from jax import lax
from jax.experimental import pallas as pl
from jax.experimental.pallas import tpu as pltpu
from jax.experimental.pallas import tpu_sc as plsc

N = 10000
E = 320000
D = 128
DH = D // 2
EPS = 1e-5

NC = 2
NS = 16

CHUNK = 80
EDGES_PER_TILE = E // NS
CHUNKS_PER_TILE = EDGES_PER_TILE // CHUNK

ROWS_MAIN = 624
TAIL_BASE = ROWS_MAIN * NS
TAIL = N - TAIL_BASE

_MESH = plsc.VectorSubcoreMesh(
    core_axis_name="c", subcore_axis_name="s", num_cores=NC, num_subcores=NS
)


def _rows_copy(src, dst, s, stage, tailbuf):
    base = pl.multiple_of(s * ROWS_MAIN, 8)
    pltpu.sync_copy(src.at[pl.ds(base, ROWS_MAIN)], stage)
    pltpu.sync_copy(stage, dst.at[pl.ds(base, ROWS_MAIN)])

    @pl.when(s == NS - 1)
    def _():
        pltpu.sync_copy(src.at[pl.ds(TAIL_BASE, TAIL)],
                        tailbuf.at[pl.ds(0, TAIL)])
        pltpu.sync_copy(tailbuf.at[pl.ds(0, TAIL)],
                        dst.at[pl.ds(TAIL_BASE, TAIL)])


@functools.partial(
    pl.kernel,
    out_type=(jax.ShapeDtypeStruct((N,), jnp.float32),
              jax.ShapeDtypeStruct((N,), jnp.float32)),
    mesh=_MESH,
    compiler_params=pltpu.CompilerParams(use_tc_tiling_on_sc=False),
    scratch_types=[
        pltpu.VMEM((CHUNK,), jnp.int32),
        pltpu.VMEM((CHUNK,), jnp.float32),
        pltpu.VMEM((ROWS_MAIN,), jnp.float32),
        pltpu.VMEM((TAIL,), jnp.float32),
        pltpu.VMEM_SHARED((N,), jnp.float32),
    ],
)
def _sc_count(src_hbm, dst_hbm, zeros_hbm, cs_out, cd_out, idx_v, ones_v,
              stage, tailbuf, cnt_sh):
    c = lax.axis_index("c")
    s = lax.axis_index("s")
    _rows_copy(zeros_hbm, cnt_sh, s, stage, tailbuf)
    for i in range(CHUNK // 16):
        ones_v[pl.ds(i * 16, 16)] = jnp.full((16,), 1.0, jnp.float32)
    plsc.subcore_barrier()

    def run(edge_ref):
        def chunk(j, _):
            base = pl.multiple_of(s * EDGES_PER_TILE + j * CHUNK, 8)
            pltpu.sync_copy(edge_ref.at[pl.ds(base, CHUNK)], idx_v)
            pltpu.sync_copy(ones_v, cnt_sh.at[idx_v], add=True)
            return ()
        lax.fori_loop(0, CHUNKS_PER_TILE, chunk, ())

    @pl.when(c == 0)
    def _():
        run(src_hbm)

    @pl.when(c == 1)
    def _():
        run(dst_hbm)

    plsc.subcore_barrier()

    @pl.when(c == 0)
    def _():
        _rows_copy(cnt_sh, cs_out, s, stage, tailbuf)

    @pl.when(c == 1)
    def _():
        _rows_copy(cnt_sh, cd_out, s, stage, tailbuf)


@functools.partial(
    pl.kernel,
    out_type=jax.ShapeDtypeStruct((NC, N, DH), jnp.float32),
    mesh=_MESH,
    compiler_params=pltpu.CompilerParams(use_tc_tiling_on_sc=False),
    scratch_types=[
        pltpu.VMEM((CHUNK,), jnp.int32),
        pltpu.VMEM((CHUNK,), jnp.int32),
        pltpu.VMEM((CHUNK, DH), jnp.float32),
        pltpu.VMEM((ROWS_MAIN, DH), jnp.float32),
        pltpu.VMEM_SHARED((N, DH), jnp.float32),
        pltpu.SemaphoreType.DMA,
    ],
)
def _sc_agg(q_hbm, src_hbm, dst_hbm, zeros_hbm, out_hbm, sidx, didx, rows,
            stage, acc_sh, sem):
    c = lax.axis_index("c")
    s = lax.axis_index("s")
    _rows_copy(zeros_hbm, acc_sh, s, stage, rows)
    plsc.subcore_barrier()

    myq = q_hbm.at[c]

    def chunk(j, _):
        base = pl.multiple_of(s * EDGES_PER_TILE + j * CHUNK, 8)
        pltpu.sync_copy(src_hbm.at[pl.ds(base, CHUNK)], sidx)
        pltpu.sync_copy(dst_hbm.at[pl.ds(base, CHUNK)], didx)
        pltpu.async_copy(myq.at[sidx], rows, sem).wait()
        pltpu.sync_copy(rows, acc_sh.at[didx], add=True)
        return ()

    lax.fori_loop(0, CHUNKS_PER_TILE, chunk, ())
    plsc.subcore_barrier()
    _rows_copy(acc_sh, out_hbm.at[c], s, stage, rows)


def _inv_sqrt_deg(cnt_ref):
    return lax.rsqrt(jnp.maximum(cnt_ref[...], 1.0))


def _split_store(q_ref, h):
    q_ref[0] = h[:, :DH]
    q_ref[1] = h[:, DH:]


def _tc_first_body(feat_ref, w_ref, co_ref, q_ref):
    inv_out = _inv_sqrt_deg(co_ref)
    h = jnp.dot(feat_ref[...], w_ref[...],
                preferred_element_type=jnp.float32) * inv_out
    _split_store(q_ref, h)


def _tc_mid_body(g_ref, co_ref, ci_ref, gamma_ref, beta_ref, w_ref, q_ref, *,
                 matmul):
    inv_in = _inv_sqrt_deg(ci_ref)
    inv_out = _inv_sqrt_deg(co_ref)
    g = jnp.concatenate([g_ref[0], g_ref[1]], axis=1)
    z = g * inv_in
    m = jnp.mean(z, axis=0, keepdims=True)
    d = z - m
    v = jnp.mean(d * d, axis=0, keepdims=True)
    h = d * lax.rsqrt(v + EPS) * gamma_ref[...] + beta_ref[...]
    h = jnp.maximum(h, 0.0)
    if matmul:
        h = jnp.dot(h, w_ref[...], preferred_element_type=jnp.float32)
    _split_store(q_ref, h * inv_out)


def _tc_premul_body(g_ref, co_ref, ci_ref, gamma_ref, beta_ref, q_ref):
    _tc_mid_body(g_ref, co_ref, ci_ref, gamma_ref, beta_ref, None, q_ref,
                 matmul=False)


def _tc_last_body(g_ref, ci_ref, w_ref, b_ref, out_ref):
    inv_in = _inv_sqrt_deg(ci_ref)
    g = jnp.concatenate([g_ref[0], g_ref[1]], axis=1)
    z = g * inv_in
    out_ref[...] = jnp.dot(z, w_ref[...],
                           preferred_element_type=jnp.float32) + b_ref[...]


_SPLIT = jax.ShapeDtypeStruct((NC, N, DH), jnp.float32)

_tc_first = pl.pallas_call(_tc_first_body, out_shape=_SPLIT)

_tc_mid = pl.pallas_call(
    functools.partial(_tc_mid_body, matmul=True), out_shape=_SPLIT)

_tc_premul = pl.pallas_call(_tc_premul_body, out_shape=_SPLIT)

_tc_last = pl.pallas_call(
    _tc_last_body, out_shape=jax.ShapeDtypeStruct((N, 40), jnp.float32))


def kernel(feat, edge_index, W0, W1, W2, b2, gamma0, beta0, gamma1, beta1):
    src = edge_index[0]
    dst = edge_index[1]
    zeros1 = jnp.zeros((N,), jnp.float32)
    zerosH = jnp.zeros((N, DH), jnp.float32)

    cs, cd = _sc_count(src, dst, zeros1)
    co = cs.reshape(N, 1)
    ci = cd.reshape(N, 1)
    q0 = _tc_first(feat, W0, co)
    g0 = _sc_agg(q0, src, dst, zerosH)
    q1 = _tc_mid(g0, co, ci, gamma0, beta0, W1)
    g1 = _sc_agg(q1, src, dst, zerosH)
    y2 = _tc_premul(g1, co, ci, gamma1, beta1)
    g2 = _sc_agg(y2, src, dst, zerosH)
    out = _tc_last(g2, ci, W2, b2)
    return out


# speedup vs baseline: 6.1656x; 2.1348x over previous
import functools

import jax
import jax.numpy as jnp
from jax import lax
from jax.experimental import pallas as pl
from jax.experimental.pallas import tpu as pltpu
from jax.experimental.pallas import tpu_sc as plsc

N = 10000
E = 320000
D = 128
DH = D // 2
EPS = 1e-5

NC = 2
NS = 16

CHUNK = 80
EDGES_PER_TILE = E // NS
CHUNKS_PER_TILE = EDGES_PER_TILE // CHUNK

ROWS_MAIN = 624
TAIL_BASE = ROWS_MAIN * NS
TAIL = N - TAIL_BASE

_MESH = plsc.VectorSubcoreMesh(
    core_axis_name="c", subcore_axis_name="s", num_cores=NC, num_subcores=NS
)


def _rows_copy(src, dst, s, stage, tailbuf):
    base = pl.multiple_of(s * ROWS_MAIN, 8)
    pltpu.sync_copy(src.at[pl.ds(base, ROWS_MAIN)], stage)
    pltpu.sync_copy(stage, dst.at[pl.ds(base, ROWS_MAIN)])

    @pl.when(s == NS - 1)
    def _():
        pltpu.sync_copy(src.at[pl.ds(TAIL_BASE, TAIL)],
                        tailbuf.at[pl.ds(0, TAIL)])
        pltpu.sync_copy(tailbuf.at[pl.ds(0, TAIL)],
                        dst.at[pl.ds(TAIL_BASE, TAIL)])


@functools.partial(
    pl.kernel,
    out_type=(jax.ShapeDtypeStruct((N,), jnp.float32),
              jax.ShapeDtypeStruct((N,), jnp.float32)),
    mesh=_MESH,
    compiler_params=pltpu.CompilerParams(use_tc_tiling_on_sc=False),
    scratch_types=[
        pltpu.VMEM((CHUNKS_PER_TILE, CHUNK), jnp.int32),
        pltpu.VMEM((CHUNK,), jnp.float32),
        pltpu.VMEM((ROWS_MAIN,), jnp.float32),
        pltpu.VMEM((TAIL,), jnp.float32),
        pltpu.VMEM_SHARED((N,), jnp.float32),
        pltpu.SemaphoreType.DMA,
    ],
)
def _sc_count(src_hbm, dst_hbm, zeros_hbm, cs_out, cd_out, ibuf, ones_v,
              stage, tailbuf, cnt_sh, sem):
    c = lax.axis_index("c")
    s = lax.axis_index("s")
    _rows_copy(zeros_hbm, cnt_sh, s, stage, tailbuf)
    for i in range(CHUNK // 16):
        ones_v[pl.ds(i * 16, 16)] = jnp.full((16,), 1.0, jnp.float32)
    rbase = s * CHUNKS_PER_TILE

    def run(edge_ref):
        pltpu.sync_copy(edge_ref.at[pl.ds(rbase, CHUNKS_PER_TILE)], ibuf)
        plsc.subcore_barrier()

        k = 10

        def group(g, _):
            for b in range(k):
                pltpu.async_copy(ones_v, cnt_sh.at[ibuf.at[g * k + b]], sem,
                                 add=True)
            for b in range(k):
                pltpu.make_async_copy(
                    ones_v, cnt_sh.at[ibuf.at[g * k + b]], sem).wait()
            return ()

        lax.fori_loop(0, CHUNKS_PER_TILE // k, group, ())

    @pl.when(c == 0)
    def _():
        run(src_hbm)

    @pl.when(c == 1)
    def _():
        run(dst_hbm)

    plsc.subcore_barrier()

    @pl.when(c == 0)
    def _():
        _rows_copy(cnt_sh, cs_out, s, stage, tailbuf)

    @pl.when(c == 1)
    def _():
        _rows_copy(cnt_sh, cd_out, s, stage, tailbuf)


@functools.partial(
    pl.kernel,
    out_type=jax.ShapeDtypeStruct((NC, N, DH), jnp.float32),
    mesh=_MESH,
    compiler_params=pltpu.CompilerParams(use_tc_tiling_on_sc=False),
    scratch_types=[
        pltpu.VMEM((CHUNKS_PER_TILE, CHUNK), jnp.int32),
        pltpu.VMEM((CHUNKS_PER_TILE, CHUNK), jnp.int32),
        pltpu.VMEM((CHUNK, DH), jnp.float32),
        pltpu.VMEM((CHUNK, DH), jnp.float32),
        pltpu.VMEM((ROWS_MAIN, DH), jnp.float32),
        pltpu.VMEM_SHARED((N, DH), jnp.float32),
        pltpu.SemaphoreType.DMA,
    ],
)
def _sc_agg(q_hbm, src_hbm, dst_hbm, zeros_hbm, out_hbm, sbuf, dbuf, rows0,
            rows1, stage, acc_sh, sem):
    c = lax.axis_index("c")
    s = lax.axis_index("s")
    _rows_copy(zeros_hbm, acc_sh, s, stage, rows0)
    rbase = s * CHUNKS_PER_TILE
    pltpu.sync_copy(src_hbm.at[pl.ds(rbase, CHUNKS_PER_TILE)], sbuf)
    pltpu.sync_copy(dst_hbm.at[pl.ds(rbase, CHUNKS_PER_TILE)], dbuf)
    plsc.subcore_barrier()

    myq = q_hbm.at[c]
    rows = (rows0, rows1)

    pltpu.async_copy(myq.at[sbuf.at[0]], rows0, sem)

    def group(g, _):
        for b in range(2):
            t = 2 * g + b
            pltpu.make_async_copy(myq.at[sbuf.at[t]], rows[b], sem).wait()

            @pl.when(t + 1 < CHUNKS_PER_TILE)
            def _():
                pltpu.async_copy(myq.at[sbuf.at[t + 1]], rows[1 - b], sem)

            pltpu.sync_copy(rows[b], acc_sh.at[dbuf.at[t]], add=True)
        return ()

    lax.fori_loop(0, CHUNKS_PER_TILE // 2, group, ())
    plsc.subcore_barrier()
    _rows_copy(acc_sh, out_hbm.at[c], s, stage, rows0)


def _inv_sqrt_deg(cnt_ref):
    return lax.rsqrt(jnp.maximum(cnt_ref[...], 1.0))


def _split_store(q_ref, h):
    q_ref[0] = h[:, :DH]
    q_ref[1] = h[:, DH:]


def _tc_first_body(feat_ref, w_ref, co_ref, q_ref):
    inv_out = _inv_sqrt_deg(co_ref)
    h = jnp.dot(feat_ref[...], w_ref[...],
                preferred_element_type=jnp.float32) * inv_out
    _split_store(q_ref, h)


def _tc_mid_body(g_ref, co_ref, ci_ref, gamma_ref, beta_ref, w_ref, q_ref, *,
                 matmul):
    inv_in = _inv_sqrt_deg(ci_ref)
    inv_out = _inv_sqrt_deg(co_ref)
    g = jnp.concatenate([g_ref[0], g_ref[1]], axis=1)
    z = g * inv_in
    m = jnp.mean(z, axis=0, keepdims=True)
    d = z - m
    v = jnp.mean(d * d, axis=0, keepdims=True)
    h = d * lax.rsqrt(v + EPS) * gamma_ref[...] + beta_ref[...]
    h = jnp.maximum(h, 0.0)
    if matmul:
        h = jnp.dot(h, w_ref[...], preferred_element_type=jnp.float32)
    _split_store(q_ref, h * inv_out)


def _tc_premul_body(g_ref, co_ref, ci_ref, gamma_ref, beta_ref, q_ref):
    _tc_mid_body(g_ref, co_ref, ci_ref, gamma_ref, beta_ref, None, q_ref,
                 matmul=False)


def _tc_last_body(g_ref, ci_ref, w_ref, b_ref, out_ref):
    inv_in = _inv_sqrt_deg(ci_ref)
    g = jnp.concatenate([g_ref[0], g_ref[1]], axis=1)
    z = g * inv_in
    out_ref[...] = jnp.dot(z, w_ref[...],
                           preferred_element_type=jnp.float32) + b_ref[...]


_SPLIT = jax.ShapeDtypeStruct((NC, N, DH), jnp.float32)

_tc_first = pl.pallas_call(_tc_first_body, out_shape=_SPLIT)

_tc_mid = pl.pallas_call(
    functools.partial(_tc_mid_body, matmul=True), out_shape=_SPLIT)

_tc_premul = pl.pallas_call(_tc_premul_body, out_shape=_SPLIT)

_tc_last = pl.pallas_call(
    _tc_last_body, out_shape=jax.ShapeDtypeStruct((N, 40), jnp.float32))


def kernel(feat, edge_index, W0, W1, W2, b2, gamma0, beta0, gamma1, beta1):
    src = edge_index[0].reshape(E // CHUNK, CHUNK)
    dst = edge_index[1].reshape(E // CHUNK, CHUNK)
    zeros1 = jnp.zeros((N,), jnp.float32)
    zerosH = jnp.zeros((N, DH), jnp.float32)

    cs, cd = _sc_count(src, dst, zeros1)
    co = cs.reshape(N, 1)
    ci = cd.reshape(N, 1)
    q0 = _tc_first(feat, W0, co)
    g0 = _sc_agg(q0, src, dst, zerosH)
    q1 = _tc_mid(g0, co, ci, gamma0, beta0, W1)
    g1 = _sc_agg(q1, src, dst, zerosH)
    y2 = _tc_premul(g1, co, ci, gamma1, beta1)
    g2 = _sc_agg(y2, src, dst, zerosH)
    out = _tc_last(g2, ci, W2, b2)
    return out
